# baseline (speedup 1.0000x reference)
import jax
import jax.numpy as jnp
from jax import lax
from jax.experimental import pallas as pl
from jax.experimental.pallas import tpu as pltpu
from jax.experimental.pallas import tpu_sc as plsc

B = 16384
F = 26
FIELD_DIM = 38461
NC, NS, L = 2, 16, 16
NW = NC * NS
RPW = B // NW
IPW = RPW * F
CHUNK = 128
NRG = RPW // CHUNK
NCHUNK = IPW // CHUNK


def _body(xtw_hbm, wrow_hbm, bias_hbm, out_hbm, xbuf, gbuf, bias_v, obuf,
          sem0, sem1, sem2, sem3):
    wid = lax.axis_index("s") * NC + lax.axis_index("c")
    sems = [sem0, sem1, sem2, sem3]

    pltpu.sync_copy(bias_hbm, bias_v)

    stages = [
        pltpu.make_async_copy(xtw_hbm.at[wid, rg], xbuf.at[rg], sems[rg])
        for rg in range(NRG)
    ]
    for st in stages:
        st.start()

    for rg in range(NRG):
        stages[rg].wait()

        def fire(f, _, rg=rg):
            for i in range(2):
                pltpu.async_copy(
                    wrow_hbm.at[xbuf.at[rg, pl.ds(f * 2 + i, 1)]],
                    gbuf.at[
                        pl.ds(0, 1),
                        pl.ds((rg * F + f * 2 + i) * CHUNK, CHUNK),
                    ],
                    sem=sems[rg],
                )
            return 0

        lax.fori_loop(0, F // 2, fire, 0)

    bias_vec = bias_v[...]
    gflat = gbuf.at[0]
    for rg in range(NRG):
        pltpu.make_async_copy(
            wrow_hbm.at[pl.ds(0, 1), pl.ds(0, F * CHUNK)],
            gbuf.at[pl.ds(0, 1), pl.ds(rg * F * CHUNK, F * CHUNK)],
            sems[rg],
        ).wait()

        def reduce(j, _, rg=rg):
            base = rg * F * CHUNK + j * L
            vacc = bias_vec
            for f in range(F):
                vacc = vacc + gflat[pl.ds(base + f * CHUNK, L)]
            obuf[pl.ds(rg * CHUNK + j * L, L)] = 1.0 / (1.0 + jnp.exp(-vacc))
            return 0

        lax.fori_loop(0, CHUNK // L, reduce, 0)

        pltpu.sync_copy(
            obuf.at[pl.ds(rg * CHUNK, CHUNK)],
            out_hbm.at[pl.ds(wid * RPW + rg * CHUNK, CHUNK)],
        )


def _prep_body(x_ref, o_ref):
    xb = x_ref[...].astype(jnp.float32)
    eye = jnp.eye(F, dtype=jnp.float32)
    t = lax.dot_general(
        eye, xb,
        dimension_numbers=(((1,), (1,)), ((), ())),
        preferred_element_type=jnp.float32,
        precision=lax.Precision.HIGHEST,
    )
    offs = (jnp.arange(F, dtype=jnp.int32) * FIELD_DIM)[:, None]
    o_ref[0] = t.astype(jnp.int32) + offs


@jax.jit
def kernel(x, weight, bias):
    xtw = pl.pallas_call(
        _prep_body,
        grid=(NW * NRG,),
        in_specs=[pl.BlockSpec((CHUNK, F), lambda i: (i, 0))],
        out_specs=pl.BlockSpec((1, F, CHUNK), lambda i: (i, 0, 0)),
        out_shape=jax.ShapeDtypeStruct((NW * NRG, F, CHUNK), jnp.int32),
    )(x)
    xtw = xtw.reshape(NW, NRG, F, CHUNK)
    bias16 = jnp.broadcast_to(bias, (L,))

    mesh = plsc.VectorSubcoreMesh(core_axis_name="c", subcore_axis_name="s")
    run = pl.kernel(
        _body,
        out_type=jax.ShapeDtypeStruct((B,), jnp.float32),
        mesh=mesh,
        scratch_types=[
            pltpu.VMEM((NRG, F, CHUNK), jnp.int32),
            pltpu.VMEM((1, IPW), jnp.float32),
            pltpu.VMEM((L,), jnp.float32),
            pltpu.VMEM((RPW,), jnp.float32),
            pltpu.SemaphoreType.DMA,
            pltpu.SemaphoreType.DMA,
            pltpu.SemaphoreType.DMA,
            pltpu.SemaphoreType.DMA,
        ],
    )
    return run(xtw, weight.reshape(1, -1), bias16)


# speedup vs baseline: 1.8953x; 1.8953x over previous
import jax
import jax.numpy as jnp
from jax import lax
from jax.experimental import pallas as pl
from jax.experimental.pallas import tpu as pltpu
from jax.experimental.pallas import tpu_sc as plsc

B = 16384
F = 26
FIELD_DIM = 38461
NC, NS, L = 2, 16, 16
NW = NC * NS
RPW = B // NW
IPW = RPW * F
CHUNK = 128
NRG = RPW // CHUNK
NCHUNK = IPW // CHUNK


def _body(xtw_hbm, wrow_hbm, bias_hbm, out_hbm, xbuf, gbuf, bias_v, obuf,
          sem0, sem1, sem2, sem3):
    wid = lax.axis_index("s") * NC + lax.axis_index("c")
    sems = [sem0, sem1, sem2, sem3]

    pltpu.sync_copy(bias_hbm, bias_v)

    stages = [
        pltpu.make_async_copy(xtw_hbm.at[wid, rg], xbuf.at[rg], sems[rg])
        for rg in range(NRG)
    ]
    for st in stages:
        st.start()

    for rg in range(NRG):
        stages[rg].wait()

        def fire(f, _, rg=rg):
            for i in range(2):
                pltpu.async_copy(
                    wrow_hbm.at[xbuf.at[rg, pl.ds(f * 2 + i, 1)]],
                    gbuf.at[
                        pl.ds(0, 1),
                        pl.ds((rg * F + f * 2 + i) * CHUNK, CHUNK),
                    ],
                    sem=sems[rg],
                )
            return 0

        lax.fori_loop(0, F // 2, fire, 0)

    bias_vec = bias_v[...]
    gflat = gbuf.at[0]
    for rg in range(NRG):
        pltpu.make_async_copy(
            wrow_hbm.at[pl.ds(0, 1), pl.ds(0, F * CHUNK)],
            gbuf.at[pl.ds(0, 1), pl.ds(rg * F * CHUNK, F * CHUNK)],
            sems[rg],
        ).wait()

        def reduce(j, _, rg=rg):
            base = rg * F * CHUNK + j * L
            vacc = bias_vec
            for f in range(F):
                vacc = vacc + gflat[pl.ds(base + f * CHUNK, L)]
            obuf[pl.ds(rg * CHUNK + j * L, L)] = 1.0 / (1.0 + jnp.exp(-vacc))
            return 0

        lax.fori_loop(0, CHUNK // L, reduce, 0)

        pltpu.sync_copy(
            obuf.at[pl.ds(rg * CHUNK, CHUNK)],
            out_hbm.at[pl.ds(wid * RPW + rg * CHUNK, CHUNK)],
        )


SUB = 8


def _prep_body(x_ref, o_ref):
    eye = jnp.eye(F, dtype=jnp.float32)
    offs = (jnp.arange(F, dtype=jnp.int32) * FIELD_DIM)[:, None]
    for s in range(SUB):
        xb = x_ref[pl.ds(s * CHUNK, CHUNK), :].astype(jnp.float32)
        t = lax.dot_general(
            eye, xb,
            dimension_numbers=(((1,), (1,)), ((), ())),
            preferred_element_type=jnp.float32,
            precision=lax.Precision.HIGHEST,
        )
        o_ref[s] = t.astype(jnp.int32) + offs


@jax.jit
def kernel(x, weight, bias):
    xtw = pl.pallas_call(
        _prep_body,
        grid=(NW * NRG // SUB,),
        in_specs=[pl.BlockSpec((SUB * CHUNK, F), lambda i: (i, 0))],
        out_specs=pl.BlockSpec((SUB, F, CHUNK), lambda i: (i, 0, 0)),
        out_shape=jax.ShapeDtypeStruct((NW * NRG, F, CHUNK), jnp.int32),
    )(x)
    xtw = xtw.reshape(NW, NRG, F, CHUNK)
    bias16 = jnp.broadcast_to(bias, (L,))

    mesh = plsc.VectorSubcoreMesh(core_axis_name="c", subcore_axis_name="s")
    run = pl.kernel(
        _body,
        out_type=jax.ShapeDtypeStruct((B,), jnp.float32),
        mesh=mesh,
        scratch_types=[
            pltpu.VMEM((NRG, F, CHUNK), jnp.int32),
            pltpu.VMEM((1, IPW), jnp.float32),
            pltpu.VMEM((L,), jnp.float32),
            pltpu.VMEM((RPW,), jnp.float32),
            pltpu.SemaphoreType.DMA,
            pltpu.SemaphoreType.DMA,
            pltpu.SemaphoreType.DMA,
            pltpu.SemaphoreType.DMA,
        ],
    )
    return run(xtw, weight.reshape(1, -1), bias16)


# speedup vs baseline: 2.6602x; 1.4035x over previous
import jax
import jax.numpy as jnp
from jax import lax
from jax.experimental import pallas as pl
from jax.experimental.pallas import tpu as pltpu
from jax.experimental.pallas import tpu_sc as plsc

B = 16384
F = 26
FIELD_DIM = 38461
NC, NS, L = 2, 16, 16
NW = NC * NS
RPW = B // NW
IPW = RPW * F
CHUNK = 128
NRG = RPW // CHUNK
NCHUNK = IPW // CHUNK


def _body(xtw_hbm, wrow_hbm, bias_hbm, out_hbm, xbuf, gbuf, bias_v, obuf,
          sem0, sem1, sem2, sem3):
    wid = lax.axis_index("s") * NC + lax.axis_index("c")
    sems = [sem0, sem1, sem2, sem3]

    pltpu.sync_copy(bias_hbm, bias_v)

    stages = [
        pltpu.make_async_copy(xtw_hbm.at[wid, rg], xbuf.at[rg], sems[rg])
        for rg in range(NRG)
    ]
    for st in stages:
        st.start()

    for rg in range(NRG):
        stages[rg].wait()

        def fire(f, _, rg=rg):
            for i in range(2):
                pltpu.async_copy(
                    wrow_hbm.at[xbuf.at[rg, pl.ds(f * 2 + i, 1)]],
                    gbuf.at[
                        pl.ds(0, 1),
                        pl.ds((rg * F + f * 2 + i) * CHUNK, CHUNK),
                    ],
                    sem=sems[rg],
                )
            return 0

        lax.fori_loop(0, F // 2, fire, 0)

    bias_vec = bias_v[...]
    gflat = gbuf.at[0]
    for rg in range(NRG):
        pltpu.make_async_copy(
            wrow_hbm.at[pl.ds(0, 1), pl.ds(0, F * CHUNK)],
            gbuf.at[pl.ds(0, 1), pl.ds(rg * F * CHUNK, F * CHUNK)],
            sems[rg],
        ).wait()

        def reduce(j, _, rg=rg):
            base = rg * F * CHUNK + j * L
            vacc = bias_vec
            for f in range(F):
                vacc = vacc + gflat[pl.ds(base + f * CHUNK, L)]
            obuf[pl.ds(rg * CHUNK + j * L, L)] = 1.0 / (1.0 + jnp.exp(-vacc))
            return 0

        lax.fori_loop(0, CHUNK // L, reduce, 0)

        pltpu.sync_copy(
            obuf.at[pl.ds(rg * CHUNK, CHUNK)],
            out_hbm.at[pl.ds(wid * RPW + rg * CHUNK, CHUNK)],
        )


@jax.jit
def kernel(x, weight, bias):
    offs = jnp.arange(F, dtype=jnp.float32) * FIELD_DIM
    lhs = jnp.broadcast_to(jnp.eye(F, dtype=jnp.float32), (NW, NRG, F, F))
    rhs = x.reshape(NW, NRG, CHUNK, F).astype(jnp.float32)
    xt = lax.dot_general(
        lhs, rhs,
        dimension_numbers=(((3,), (3,)), ((0, 1), (0, 1))),
        preferred_element_type=jnp.float32,
        precision=lax.Precision.HIGHEST,
    ) + offs[None, None, :, None]
    xtw = xt.astype(jnp.int32)
    bias16 = jnp.broadcast_to(bias, (L,))

    mesh = plsc.VectorSubcoreMesh(core_axis_name="c", subcore_axis_name="s")
    run = pl.kernel(
        _body,
        out_type=jax.ShapeDtypeStruct((B,), jnp.float32),
        mesh=mesh,
        scratch_types=[
            pltpu.VMEM((NRG, F, CHUNK), jnp.int32),
            pltpu.VMEM((1, IPW), jnp.float32),
            pltpu.VMEM((L,), jnp.float32),
            pltpu.VMEM((RPW,), jnp.float32),
            pltpu.SemaphoreType.DMA,
            pltpu.SemaphoreType.DMA,
            pltpu.SemaphoreType.DMA,
            pltpu.SemaphoreType.DMA,
        ],
    )
    return run(xtw, weight.reshape(1, -1), bias16)


# speedup vs baseline: 2.7753x; 1.0433x over previous
import jax
import jax.numpy as jnp
from jax import lax
from jax.experimental import pallas as pl
from jax.experimental.pallas import tpu as pltpu
from jax.experimental.pallas import tpu_sc as plsc

B = 16384
F = 26
FIELD_DIM = 38461
NC, NS, L = 2, 16, 16
NW = NC * NS
RPW = B // NW
IPW = RPW * F
CHUNK = 128
NRG = RPW // CHUNK
NCHUNK = IPW // CHUNK


def _body(xtw_hbm, wrow_hbm, bias_hbm, out_hbm, xbuf, gbuf, bias_v, obuf,
          sem0, sem1, sem2, sem3):
    wid = lax.axis_index("s") * NC + lax.axis_index("c")
    sems = [sem0, sem1, sem2, sem3]

    pltpu.sync_copy(bias_hbm, bias_v)

    stages = [
        pltpu.make_async_copy(xtw_hbm.at[:, wid, rg], xbuf.at[rg], sems[rg])
        for rg in range(NRG)
    ]
    for st in stages:
        st.start()

    for rg in range(NRG):
        stages[rg].wait()

        def fire(f, _, rg=rg):
            for i in range(2):
                pltpu.async_copy(
                    wrow_hbm.at[xbuf.at[rg, pl.ds(f * 2 + i, 1)]],
                    gbuf.at[
                        pl.ds(0, 1),
                        pl.ds((rg * F + f * 2 + i) * CHUNK, CHUNK),
                    ],
                    sem=sems[rg],
                )
            return 0

        lax.fori_loop(0, F // 2, fire, 0)

    bias_vec = bias_v[...]
    gflat = gbuf.at[0]
    for rg in range(NRG):
        pltpu.make_async_copy(
            wrow_hbm.at[pl.ds(0, 1), pl.ds(0, F * CHUNK)],
            gbuf.at[pl.ds(0, 1), pl.ds(rg * F * CHUNK, F * CHUNK)],
            sems[rg],
        ).wait()

        def reduce(j, _, rg=rg):
            base = rg * F * CHUNK + j * L
            vacc = bias_vec
            for f in range(F):
                vacc = vacc + gflat[pl.ds(base + f * CHUNK, L)]
            obuf[pl.ds(rg * CHUNK + j * L, L)] = 1.0 / (1.0 + jnp.exp(-vacc))
            return 0

        lax.fori_loop(0, CHUNK // L, reduce, 0)

        pltpu.sync_copy(
            obuf.at[pl.ds(rg * CHUNK, CHUNK)],
            out_hbm.at[pl.ds(wid * RPW + rg * CHUNK, CHUNK)],
        )


@jax.jit
def kernel(x, weight, bias):
    offs = jnp.arange(F, dtype=jnp.float32) * FIELD_DIM
    xt = lax.dot_general(
        jnp.eye(F, dtype=jnp.float32), x.astype(jnp.float32),
        dimension_numbers=(((1,), (1,)), ((), ())),
        preferred_element_type=jnp.float32,
        precision=lax.Precision.HIGHEST,
    ) + offs[:, None]
    xtw = xt.astype(jnp.int32).reshape(F, NW, NRG, CHUNK)
    bias16 = jnp.broadcast_to(bias, (L,))

    mesh = plsc.VectorSubcoreMesh(core_axis_name="c", subcore_axis_name="s")
    run = pl.kernel(
        _body,
        out_type=jax.ShapeDtypeStruct((B,), jnp.float32),
        mesh=mesh,
        scratch_types=[
            pltpu.VMEM((NRG, F, CHUNK), jnp.int32),
            pltpu.VMEM((1, IPW), jnp.float32),
            pltpu.VMEM((L,), jnp.float32),
            pltpu.VMEM((RPW,), jnp.float32),
            pltpu.SemaphoreType.DMA,
            pltpu.SemaphoreType.DMA,
            pltpu.SemaphoreType.DMA,
            pltpu.SemaphoreType.DMA,
        ],
    )
    return run(xtw, weight.reshape(1, -1), bias16)


# speedup vs baseline: 2.8496x; 1.0267x over previous
import jax
import jax.numpy as jnp
from jax import lax
from jax.experimental import pallas as pl
from jax.experimental.pallas import tpu as pltpu
from jax.experimental.pallas import tpu_sc as plsc

B = 16384
F = 26
FIELD_DIM = 38461
NC, NS, L = 2, 16, 16
NW = NC * NS
RPW = B // NW
IPW = RPW * F
CHUNK = 128
NRG = RPW // CHUNK
NCHUNK = IPW // CHUNK


def _body(xtw_hbm, wrow_hbm, bias_hbm, out_hbm, xbuf, gbuf, bias_v, obuf,
          sem0, sem1, sem2, sem3):
    wid = lax.axis_index("s") * NC + lax.axis_index("c")
    sems = [sem0, sem1, sem2, sem3]

    pltpu.sync_copy(bias_hbm, bias_v)

    stages = [
        pltpu.make_async_copy(
            xtw_hbm.at[:, pl.ds(wid * RPW + rg * CHUNK, CHUNK)],
            xbuf.at[rg],
            sems[rg],
        )
        for rg in range(NRG)
    ]
    for st in stages:
        st.start()

    for rg in range(NRG):
        stages[rg].wait()

        def fire(f, _, rg=rg):
            for i in range(2):
                pltpu.async_copy(
                    wrow_hbm.at[xbuf.at[rg, pl.ds(f * 2 + i, 1)]],
                    gbuf.at[
                        pl.ds(0, 1),
                        pl.ds((rg * F + f * 2 + i) * CHUNK, CHUNK),
                    ],
                    sem=sems[rg],
                )
            return 0

        lax.fori_loop(0, F // 2, fire, 0)

    bias_vec = bias_v[...]
    gflat = gbuf.at[0]
    for rg in range(NRG):
        pltpu.make_async_copy(
            wrow_hbm.at[pl.ds(0, 1), pl.ds(0, F * CHUNK)],
            gbuf.at[pl.ds(0, 1), pl.ds(rg * F * CHUNK, F * CHUNK)],
            sems[rg],
        ).wait()

        def reduce(j, _, rg=rg):
            base = rg * F * CHUNK + j * L
            vacc = bias_vec
            for f in range(F):
                vacc = vacc + gflat[pl.ds(base + f * CHUNK, L)]
            obuf[pl.ds(rg * CHUNK + j * L, L)] = 1.0 / (1.0 + jnp.exp(-vacc))
            return 0

        lax.fori_loop(0, CHUNK // L, reduce, 0)

        pltpu.sync_copy(
            obuf.at[pl.ds(rg * CHUNK, CHUNK)],
            out_hbm.at[pl.ds(wid * RPW + rg * CHUNK, CHUNK)],
        )


@jax.jit
def kernel(x, weight, bias):
    offs = jnp.arange(F, dtype=jnp.float32) * FIELD_DIM
    xt = lax.dot_general(
        jnp.eye(F, dtype=jnp.float32), x.astype(jnp.float32),
        dimension_numbers=(((1,), (1,)), ((), ())),
        preferred_element_type=jnp.float32,
        precision=lax.Precision.HIGHEST,
    ) + offs[:, None]
    xtw = xt.astype(jnp.int32)
    bias16 = jnp.broadcast_to(bias, (L,))

    mesh = plsc.VectorSubcoreMesh(core_axis_name="c", subcore_axis_name="s")
    run = pl.kernel(
        _body,
        out_type=jax.ShapeDtypeStruct((B,), jnp.float32),
        mesh=mesh,
        scratch_types=[
            pltpu.VMEM((NRG, F, CHUNK), jnp.int32),
            pltpu.VMEM((1, IPW), jnp.float32),
            pltpu.VMEM((L,), jnp.float32),
            pltpu.VMEM((RPW,), jnp.float32),
            pltpu.SemaphoreType.DMA,
            pltpu.SemaphoreType.DMA,
            pltpu.SemaphoreType.DMA,
            pltpu.SemaphoreType.DMA,
        ],
    )
    return run(xtw, weight.reshape(1, -1), bias16)
